# baseline (speedup 1.0000x reference)
import jax
import jax.numpy as jnp
from jax.experimental import pallas as pl
from jax.experimental.pallas import tpu as pltpu

N, K, D = 65536, 1024, 32
BN = 1024
NB = N // BN
DA = D + 1


def _body(x_ref, ct_ref, assign_ref, cent_ref, sdist_ref,
          sums_scr, sacc_scr):
    i = pl.program_id(0)

    def _split(v):
        hi = v.astype(jnp.bfloat16)
        lo = (v - hi.astype(jnp.float32)).astype(jnp.bfloat16)
        return hi, lo

    @pl.when(i == 0)
    def _init():
        sums_scr[...] = jnp.zeros_like(sums_scr)
        sacc_scr[...] = jnp.zeros_like(sacc_scr)

    ct = ct_ref[...]
    cn = jnp.sum(ct * ct, axis=0, keepdims=True)
    ca = jnp.concatenate([-2.0 * ct, cn], axis=0)
    ca_hi, ca_lo = _split(ca)
    c3 = jnp.concatenate([ca_hi, ca_hi, ca_lo, ca_lo], axis=0)

    x = x_ref[...]
    ones_col = jnp.ones((BN, 1), jnp.float32)
    xa = jnp.concatenate([x, ones_col], axis=1)
    xa_hi, xa_lo = _split(xa)

    def _dot(a, b):
        return jax.lax.dot_general(
            a, b, dimension_numbers=(((1,), (0,)), ((), ())),
            preferred_element_type=jnp.float32)

    x3 = jnp.concatenate([xa_hi, xa_lo, xa_hi, xa_lo], axis=1)
    dist = _dot(x3, c3)

    minval = jnp.min(dist, axis=1, keepdims=True)
    iota_kf = jax.lax.broadcasted_iota(
        jnp.int32, (BN, K), 1).astype(jnp.float32)
    masked = jnp.where(dist == minval, iota_kf, jnp.float32(K))
    idx = jnp.min(masked, axis=1, keepdims=True).astype(jnp.int32)
    assign_ref[...] = idx

    idx_row = jnp.transpose(idx)
    iota_kt = jax.lax.broadcasted_iota(jnp.int32, (K, BN), 0)
    onehot_t = (iota_kt == idx_row).astype(jnp.bfloat16)
    s2 = _dot(onehot_t, x3[:, :2 * DA])
    sums_scr[...] += s2[:, :DA] + s2[:, DA:]

    xn = jnp.sum(x * x, axis=1, keepdims=True)
    sacc_scr[...] = sacc_scr[...] + jnp.sum(minval + xn)

    @pl.when(i == NB - 1)
    def _finish():
        cent_ref[...] = sums_scr[:, :D] / sums_scr[:, D:]
        sdist_ref[...] = sacc_scr[...]


@jax.jit
def kernel(input_x, input_centroids):
    assign2, cent, sdist = pl.pallas_call(
        _body,
        grid=(NB,),
        in_specs=[
            pl.BlockSpec((BN, D), lambda i: (i, 0)),
            pl.BlockSpec((D, K), lambda i: (0, 0)),
        ],
        out_specs=[
            pl.BlockSpec((BN, 1), lambda i: (i, 0)),
            pl.BlockSpec((K, D), lambda i: (0, 0)),
            pl.BlockSpec((1, 1), lambda i: (0, 0)),
        ],
        out_shape=[
            jax.ShapeDtypeStruct((N, 1), jnp.int32),
            jax.ShapeDtypeStruct((K, D), jnp.float32),
            jax.ShapeDtypeStruct((1, 1), jnp.float32),
        ],
        scratch_shapes=[
            pltpu.VMEM((K, DA), jnp.float32),
            pltpu.VMEM((1, 1), jnp.float32),
        ],
    )(input_x, input_centroids.T)
    assignments = assign2.reshape(N)
    return assignments, cent, sdist[0, 0]


# speedup vs baseline: 1.0077x; 1.0077x over previous
import jax
import jax.numpy as jnp
from jax.experimental import pallas as pl
from jax.experimental.pallas import tpu as pltpu

N, K, D = 65536, 1024, 32
BN = 1024
NB = N // BN
DA = D + 1


def _body(x_ref, ct_ref, assign_ref, cent_ref, sdist_ref,
          sums_scr, sacc_scr):
    i = pl.program_id(0)

    def _split(v):
        hi = v.astype(jnp.bfloat16)
        lo = (v - hi.astype(jnp.float32)).astype(jnp.bfloat16)
        return hi, lo

    @pl.when(i == 0)
    def _init():
        sums_scr[...] = jnp.zeros_like(sums_scr)
        sacc_scr[...] = jnp.zeros_like(sacc_scr)

    ct = ct_ref[...]
    cn = jnp.sum(ct * ct, axis=0, keepdims=True)
    ca = jnp.concatenate([-2.0 * ct, cn], axis=0)
    ca_hi, ca_lo = _split(ca)
    c3 = jnp.concatenate([ca_hi, ca_lo, ca_hi, ca_lo], axis=0)

    x = x_ref[...]
    ones_col = jnp.ones((BN, 1), jnp.float32)
    xa = jnp.concatenate([x, ones_col], axis=1)
    xa_hi, xa_lo = _split(xa)

    def _dot(a, b):
        return jax.lax.dot_general(
            a, b, dimension_numbers=(((1,), (0,)), ((), ())),
            preferred_element_type=jnp.float32)

    x3 = jnp.concatenate([xa_hi, xa_hi, xa_lo, xa_lo], axis=1)
    dist = _dot(x3, c3)

    minval = jnp.min(dist, axis=1, keepdims=True)
    iota_kf = jax.lax.broadcasted_iota(
        jnp.int32, (BN, K), 1).astype(jnp.float32)
    masked = jnp.where(dist == minval, iota_kf, jnp.float32(K))
    idx = jnp.min(masked, axis=1, keepdims=True).astype(jnp.int32)
    assign_ref[...] = idx

    idx_row = jnp.transpose(idx)
    iota_kt = jax.lax.broadcasted_iota(jnp.int32, (K, BN), 0)
    onehot_t = (iota_kt == idx_row).astype(jnp.bfloat16)
    xa2 = jnp.concatenate([xa_hi, xa_lo], axis=1)
    s2 = _dot(onehot_t, xa2)
    sums_scr[...] += s2[:, :DA] + s2[:, DA:]

    xn = jnp.sum(x * x, axis=1, keepdims=True)
    sacc_scr[...] = sacc_scr[...] + jnp.sum(minval + xn)

    @pl.when(i == NB - 1)
    def _finish():
        cent_ref[...] = sums_scr[:, :D] / sums_scr[:, D:]
        sdist_ref[...] = sacc_scr[...]


@jax.jit
def kernel(input_x, input_centroids):
    assign2, cent, sdist = pl.pallas_call(
        _body,
        grid=(NB,),
        in_specs=[
            pl.BlockSpec((BN, D), lambda i: (i, 0)),
            pl.BlockSpec((D, K), lambda i: (0, 0)),
        ],
        out_specs=[
            pl.BlockSpec((BN, 1), lambda i: (i, 0)),
            pl.BlockSpec((K, D), lambda i: (0, 0)),
            pl.BlockSpec((1, 1), lambda i: (0, 0)),
        ],
        out_shape=[
            jax.ShapeDtypeStruct((N, 1), jnp.int32),
            jax.ShapeDtypeStruct((K, D), jnp.float32),
            jax.ShapeDtypeStruct((1, 1), jnp.float32),
        ],
        scratch_shapes=[
            pltpu.VMEM((K, DA), jnp.float32),
            pltpu.VMEM((1, 1), jnp.float32),
        ],
    )(input_x, input_centroids.T)
    assignments = assign2.reshape(N)
    return assignments, cent, sdist[0, 0]


# speedup vs baseline: 1.0539x; 1.0459x over previous
import jax
import jax.numpy as jnp
from jax import lax
from jax.experimental import pallas as pl
from jax.experimental.pallas import tpu as pltpu
from jax.experimental.pallas import tpu_sc as plsc

N, K, D = 65536, 1024, 32
BN = 1024
NB = N // BN
DA = D + 1

NC, NS, L = 2, 16, 16
NW = NC * NS
RPW = N // NW
CH = 128
NCH = RPW // CH
CW = 8


def _assign_body(x_ref, ct_ref, assign_ref, sdist_ref, sacc_scr):
    i = pl.program_id(0)

    def _split(v):
        hi = v.astype(jnp.bfloat16)
        lo = (v - hi.astype(jnp.float32)).astype(jnp.bfloat16)
        return hi, lo

    @pl.when(i == 0)
    def _init():
        sacc_scr[...] = jnp.zeros_like(sacc_scr)

    ct = ct_ref[...]
    cn = jnp.sum(ct * ct, axis=0, keepdims=True)
    ca = jnp.concatenate([-2.0 * ct, cn], axis=0)
    ca_hi, ca_lo = _split(ca)
    c3 = jnp.concatenate([ca_hi, ca_lo, ca_hi, ca_lo], axis=0)

    x = x_ref[...]
    ones_col = jnp.ones((BN, 1), jnp.float32)
    xa = jnp.concatenate([x, ones_col], axis=1)
    xa_hi, xa_lo = _split(xa)

    def _dot(a, b):
        return jax.lax.dot_general(
            a, b, dimension_numbers=(((1,), (0,)), ((), ())),
            preferred_element_type=jnp.float32)

    x3 = jnp.concatenate([xa_hi, xa_hi, xa_lo, xa_lo], axis=1)
    dist = _dot(x3, c3)

    minval = jnp.min(dist, axis=1, keepdims=True)
    iota_kf = jax.lax.broadcasted_iota(
        jnp.int32, (BN, K), 1).astype(jnp.float32)
    masked = jnp.where(dist == minval, iota_kf, jnp.float32(K))
    idx = jnp.min(masked, axis=1, keepdims=True).astype(jnp.int32)
    assign_ref[...] = idx

    xn = jnp.sum(x * x, axis=1, keepdims=True)
    sacc_scr[...] = sacc_scr[...] + jnp.sum(minval + xn)

    @pl.when(i == NB - 1)
    def _finish():
        sdist_ref[...] = sacc_scr[...]


def _sc_segsum(x_hbm, idx_hbm, z_sums_hbm, z_cnt_hbm, ones_hbm,
               sums_out, cnt_out, x_v, idx_v, ones_v,
               shared_sums, shared_cnt):
    cid = lax.axis_index("c")
    sid = lax.axis_index("s")
    base = (cid * NS + sid) * RPW

    pltpu.sync_copy(ones_hbm, ones_v)

    @pl.when(sid == 0)
    def _zero():
        pltpu.sync_copy(z_sums_hbm, shared_sums)
        pltpu.sync_copy(z_cnt_hbm, shared_cnt)

    plsc.subcore_barrier()

    def _chunk(t, carry):
        b = base + t * CH
        pltpu.sync_copy(x_hbm.at[pl.ds(b, CH)], x_v)
        pltpu.sync_copy(idx_hbm.at[pl.ds(b, CH)], idx_v)
        pltpu.sync_copy(x_v, shared_sums.at[idx_v], add=True)
        pltpu.sync_copy(ones_v, shared_cnt.at[idx_v], add=True)
        return carry

    lax.fori_loop(0, NCH, _chunk, 0)
    plsc.subcore_barrier()

    @pl.when(sid == 0)
    def _flush():
        pltpu.sync_copy(shared_sums, sums_out.at[cid])
        pltpu.sync_copy(shared_cnt, cnt_out.at[cid])


_sc_segsum_call = pl.kernel(
    _sc_segsum,
    out_type=[
        jax.ShapeDtypeStruct((NC, K, D), jnp.float32),
        jax.ShapeDtypeStruct((NC, K, CW), jnp.float32),
    ],
    mesh=plsc.VectorSubcoreMesh(core_axis_name="c", subcore_axis_name="s"),
    scratch_types=[
        pltpu.VMEM((CH, D), jnp.float32),
        pltpu.VMEM((CH,), jnp.int32),
        pltpu.VMEM((CH, CW), jnp.float32),
        pltpu.VMEM_SHARED((K, D), jnp.float32),
        pltpu.VMEM_SHARED((K, CW), jnp.float32),
    ],
)


def _combine_body(sums_ref, cnt_ref, cent_ref):
    s = sums_ref[0, :, :] + sums_ref[1, :, :]
    c = cnt_ref[0, :, :1] + cnt_ref[1, :, :1]
    cent_ref[...] = s / c


@jax.jit
def kernel(input_x, input_centroids):
    assign2, sdist = pl.pallas_call(
        _assign_body,
        grid=(NB,),
        in_specs=[
            pl.BlockSpec((BN, D), lambda i: (i, 0)),
            pl.BlockSpec((D, K), lambda i: (0, 0)),
        ],
        out_specs=[
            pl.BlockSpec((BN, 1), lambda i: (i, 0)),
            pl.BlockSpec((1, 1), lambda i: (0, 0)),
        ],
        out_shape=[
            jax.ShapeDtypeStruct((N, 1), jnp.int32),
            jax.ShapeDtypeStruct((1, 1), jnp.float32),
        ],
        scratch_shapes=[
            pltpu.VMEM((1, 1), jnp.float32),
        ],
    )(input_x, input_centroids.T)
    assignments = assign2.reshape(N)

    sums_p, cnt_p = _sc_segsum_call(
        input_x,
        assignments,
        jnp.zeros((K, D), jnp.float32),
        jnp.zeros((K, CW), jnp.float32),
        jnp.ones((CH, CW), jnp.float32),
    )

    cent = pl.pallas_call(
        _combine_body,
        out_shape=jax.ShapeDtypeStruct((K, D), jnp.float32),
    )(sums_p, cnt_p)

    return assignments, cent, sdist[0, 0]


# speedup vs baseline: 1.1359x; 1.0778x over previous
import jax
import jax.numpy as jnp
from jax import lax
from jax.experimental import pallas as pl
from jax.experimental.pallas import tpu as pltpu
from jax.experimental.pallas import tpu_sc as plsc

N, K, D = 65536, 1024, 32
BN = 2048
NB = N // BN
DA = D + 1

NC, NS, L = 2, 16, 16
NW = NC * NS
RPW = N // NW
CH = 128
NCH = RPW // CH
CW = 8


def _assign_body(x_ref, ct_ref, assign_ref, sdist_ref, sacc_scr):
    i = pl.program_id(0)

    def _split(v):
        hi = v.astype(jnp.bfloat16)
        lo = (v - hi.astype(jnp.float32)).astype(jnp.bfloat16)
        return hi, lo

    @pl.when(i == 0)
    def _init():
        sacc_scr[...] = jnp.zeros_like(sacc_scr)

    ct = ct_ref[...]
    cn = jnp.sum(ct * ct, axis=0, keepdims=True)
    ca = jnp.concatenate([-2.0 * ct, cn], axis=0)
    ca_hi, ca_lo = _split(ca)
    c3 = jnp.concatenate([ca_hi, ca_lo, ca_hi, ca_lo], axis=0)

    x = x_ref[...]
    ones_col = jnp.ones((BN, 1), jnp.float32)
    xa = jnp.concatenate([x, ones_col], axis=1)
    xa_hi, xa_lo = _split(xa)

    def _dot(a, b):
        return jax.lax.dot_general(
            a, b, dimension_numbers=(((1,), (0,)), ((), ())),
            preferred_element_type=jnp.float32)

    x3 = jnp.concatenate([xa_hi, xa_hi, xa_lo, xa_lo], axis=1)
    dist = _dot(x3, c3)

    minval = jnp.min(dist, axis=1, keepdims=True)
    iota_kf = jax.lax.broadcasted_iota(
        jnp.int32, (BN, K), 1).astype(jnp.float32)
    masked = jnp.where(dist == minval, iota_kf, jnp.float32(K))
    idx = jnp.min(masked, axis=1, keepdims=True).astype(jnp.int32)
    assign_ref[...] = idx

    xn = jnp.sum(x * x, axis=1, keepdims=True)
    sacc_scr[...] = sacc_scr[...] + jnp.sum(minval + xn)

    @pl.when(i == NB - 1)
    def _finish():
        sdist_ref[...] = sacc_scr[...]


def _sc_segsum(x_hbm, idx_hbm, z_sums_hbm, z_cnt_hbm, ones_hbm,
               sums_out, cnt_out, x_v, idx_v, ones_v,
               shared_sums, shared_cnt):
    cid = lax.axis_index("c")
    sid = lax.axis_index("s")
    base = (cid * NS + sid) * RPW

    pltpu.sync_copy(ones_hbm, ones_v)

    @pl.when(sid == 0)
    def _zero():
        pltpu.sync_copy(z_sums_hbm, shared_sums)
        pltpu.sync_copy(z_cnt_hbm, shared_cnt)

    plsc.subcore_barrier()

    def _chunk(t, carry):
        b = base + t * CH
        pltpu.sync_copy(x_hbm.at[pl.ds(b, CH)], x_v)
        pltpu.sync_copy(idx_hbm.at[pl.ds(b, CH)], idx_v)
        pltpu.sync_copy(x_v, shared_sums.at[idx_v], add=True)
        pltpu.sync_copy(ones_v, shared_cnt.at[idx_v], add=True)
        return carry

    lax.fori_loop(0, NCH, _chunk, 0)
    plsc.subcore_barrier()

    @pl.when(sid == 0)
    def _flush():
        pltpu.sync_copy(shared_sums, sums_out.at[cid])
        pltpu.sync_copy(shared_cnt, cnt_out.at[cid])


_sc_segsum_call = pl.kernel(
    _sc_segsum,
    out_type=[
        jax.ShapeDtypeStruct((NC, K, D), jnp.float32),
        jax.ShapeDtypeStruct((NC, K, CW), jnp.float32),
    ],
    mesh=plsc.VectorSubcoreMesh(core_axis_name="c", subcore_axis_name="s"),
    scratch_types=[
        pltpu.VMEM((CH, D), jnp.float32),
        pltpu.VMEM((CH,), jnp.int32),
        pltpu.VMEM((CH, CW), jnp.float32),
        pltpu.VMEM_SHARED((K, D), jnp.float32),
        pltpu.VMEM_SHARED((K, CW), jnp.float32),
    ],
)


def _combine_body(sums_ref, cnt_ref, cent_ref):
    s = sums_ref[0, :, :] + sums_ref[1, :, :]
    c = cnt_ref[0, :, :1] + cnt_ref[1, :, :1]
    cent_ref[...] = s / c


@jax.jit
def kernel(input_x, input_centroids):
    assign2, sdist = pl.pallas_call(
        _assign_body,
        grid=(NB,),
        in_specs=[
            pl.BlockSpec((BN, D), lambda i: (i, 0)),
            pl.BlockSpec((D, K), lambda i: (0, 0)),
        ],
        out_specs=[
            pl.BlockSpec((BN, 1), lambda i: (i, 0)),
            pl.BlockSpec((1, 1), lambda i: (0, 0)),
        ],
        out_shape=[
            jax.ShapeDtypeStruct((N, 1), jnp.int32),
            jax.ShapeDtypeStruct((1, 1), jnp.float32),
        ],
        scratch_shapes=[
            pltpu.VMEM((1, 1), jnp.float32),
        ],
    )(input_x, input_centroids.T)
    assignments = assign2.reshape(N)

    sums_p, cnt_p = _sc_segsum_call(
        input_x,
        assignments,
        jnp.zeros((K, D), jnp.float32),
        jnp.zeros((K, CW), jnp.float32),
        jnp.ones((CH, CW), jnp.float32),
    )

    cent = pl.pallas_call(
        _combine_body,
        out_shape=jax.ShapeDtypeStruct((K, D), jnp.float32),
    )(sums_p, cnt_p)

    return assignments, cent, sdist[0, 0]


# speedup vs baseline: 1.1640x; 1.0248x over previous
import jax
import jax.numpy as jnp
from jax import lax
from jax.experimental import pallas as pl
from jax.experimental.pallas import tpu as pltpu
from jax.experimental.pallas import tpu_sc as plsc

N, K, D = 65536, 1024, 32
BN = 4096
NB = N // BN
DA = D + 1

NC, NS, L = 2, 16, 16
NW = NC * NS
RPW = N // NW
CH = 128
NCH = RPW // CH
CW = 8


def _assign_body(x_ref, ct_ref, assign_ref, sdist_ref, sacc_scr):
    i = pl.program_id(0)

    def _split(v):
        hi = v.astype(jnp.bfloat16)
        lo = (v - hi.astype(jnp.float32)).astype(jnp.bfloat16)
        return hi, lo

    @pl.when(i == 0)
    def _init():
        sacc_scr[...] = jnp.zeros_like(sacc_scr)

    ct = ct_ref[...]
    cn = jnp.sum(ct * ct, axis=0, keepdims=True)
    ca = jnp.concatenate([-2.0 * ct, cn], axis=0)
    ca_hi, ca_lo = _split(ca)
    c3 = jnp.concatenate([ca_hi, ca_lo, ca_hi, ca_lo], axis=0)

    x = x_ref[...]
    ones_col = jnp.ones((BN, 1), jnp.float32)
    xa = jnp.concatenate([x, ones_col], axis=1)
    xa_hi, xa_lo = _split(xa)

    def _dot(a, b):
        return jax.lax.dot_general(
            a, b, dimension_numbers=(((1,), (0,)), ((), ())),
            preferred_element_type=jnp.float32)

    x3 = jnp.concatenate([xa_hi, xa_hi, xa_lo, xa_lo], axis=1)
    dist = _dot(x3, c3)

    minval = jnp.min(dist, axis=1, keepdims=True)
    iota_kf = jax.lax.broadcasted_iota(
        jnp.int32, (BN, K), 1).astype(jnp.float32)
    masked = jnp.where(dist == minval, iota_kf, jnp.float32(K))
    idx = jnp.min(masked, axis=1, keepdims=True).astype(jnp.int32)
    assign_ref[...] = idx

    xn = jnp.sum(x * x, axis=1, keepdims=True)
    sacc_scr[...] = sacc_scr[...] + jnp.sum(minval + xn)

    @pl.when(i == NB - 1)
    def _finish():
        sdist_ref[...] = sacc_scr[...]


def _sc_segsum(x_hbm, idx_hbm, z_sums_hbm, z_cnt_hbm, ones_hbm,
               sums_out, cnt_out, x_v, idx_v, ones_v,
               shared_sums, shared_cnt):
    cid = lax.axis_index("c")
    sid = lax.axis_index("s")
    base = (cid * NS + sid) * RPW

    pltpu.sync_copy(ones_hbm, ones_v)

    @pl.when(sid == 0)
    def _zero():
        pltpu.sync_copy(z_sums_hbm, shared_sums)
        pltpu.sync_copy(z_cnt_hbm, shared_cnt)

    plsc.subcore_barrier()

    def _chunk(t, carry):
        b = base + t * CH
        pltpu.sync_copy(x_hbm.at[pl.ds(b, CH)], x_v)
        pltpu.sync_copy(idx_hbm.at[pl.ds(b, CH)], idx_v)
        pltpu.sync_copy(x_v, shared_sums.at[idx_v], add=True)
        pltpu.sync_copy(ones_v, shared_cnt.at[idx_v], add=True)
        return carry

    lax.fori_loop(0, NCH, _chunk, 0)
    plsc.subcore_barrier()

    @pl.when(sid == 0)
    def _flush():
        pltpu.sync_copy(shared_sums, sums_out.at[cid])
        pltpu.sync_copy(shared_cnt, cnt_out.at[cid])


_sc_segsum_call = pl.kernel(
    _sc_segsum,
    out_type=[
        jax.ShapeDtypeStruct((NC, K, D), jnp.float32),
        jax.ShapeDtypeStruct((NC, K, CW), jnp.float32),
    ],
    mesh=plsc.VectorSubcoreMesh(core_axis_name="c", subcore_axis_name="s"),
    scratch_types=[
        pltpu.VMEM((CH, D), jnp.float32),
        pltpu.VMEM((CH,), jnp.int32),
        pltpu.VMEM((CH, CW), jnp.float32),
        pltpu.VMEM_SHARED((K, D), jnp.float32),
        pltpu.VMEM_SHARED((K, CW), jnp.float32),
    ],
)


def _combine_body(sums_ref, cnt_ref, cent_ref):
    s = sums_ref[0, :, :] + sums_ref[1, :, :]
    c = cnt_ref[0, :, :1] + cnt_ref[1, :, :1]
    cent_ref[...] = s / c


@jax.jit
def kernel(input_x, input_centroids):
    assign2, sdist = pl.pallas_call(
        _assign_body,
        grid=(NB,),
        in_specs=[
            pl.BlockSpec((BN, D), lambda i: (i, 0)),
            pl.BlockSpec((D, K), lambda i: (0, 0)),
        ],
        out_specs=[
            pl.BlockSpec((BN, 1), lambda i: (i, 0)),
            pl.BlockSpec((1, 1), lambda i: (0, 0)),
        ],
        out_shape=[
            jax.ShapeDtypeStruct((N, 1), jnp.int32),
            jax.ShapeDtypeStruct((1, 1), jnp.float32),
        ],
        scratch_shapes=[
            pltpu.VMEM((1, 1), jnp.float32),
        ],
    )(input_x, input_centroids.T)
    assignments = assign2.reshape(N)

    sums_p, cnt_p = _sc_segsum_call(
        input_x,
        assignments,
        jnp.zeros((K, D), jnp.float32),
        jnp.zeros((K, CW), jnp.float32),
        jnp.ones((CH, CW), jnp.float32),
    )

    cent = pl.pallas_call(
        _combine_body,
        out_shape=jax.ShapeDtypeStruct((K, D), jnp.float32),
    )(sums_p, cnt_p)

    return assignments, cent, sdist[0, 0]


# speedup vs baseline: 1.1655x; 1.0013x over previous
import jax
import jax.numpy as jnp
from jax import lax
from jax.experimental import pallas as pl
from jax.experimental.pallas import tpu as pltpu
from jax.experimental.pallas import tpu_sc as plsc

N, K, D = 65536, 1024, 32
BN = 8192
NB = N // BN
DA = D + 1

NC, NS, L = 2, 16, 16
NW = NC * NS
RPW = N // NW
CH = 128
NCH = RPW // CH
CW = 8


def _assign_body(x_ref, ct_ref, assign_ref, sdist_ref, sacc_scr):
    i = pl.program_id(0)

    def _split(v):
        hi = v.astype(jnp.bfloat16)
        lo = (v - hi.astype(jnp.float32)).astype(jnp.bfloat16)
        return hi, lo

    @pl.when(i == 0)
    def _init():
        sacc_scr[...] = jnp.zeros_like(sacc_scr)

    ct = ct_ref[...]
    cn = jnp.sum(ct * ct, axis=0, keepdims=True)
    ca = jnp.concatenate([-2.0 * ct, cn], axis=0)
    ca_hi, ca_lo = _split(ca)
    c3 = jnp.concatenate([ca_hi, ca_lo, ca_hi, ca_lo], axis=0)

    x = x_ref[...]
    ones_col = jnp.ones((BN, 1), jnp.float32)
    xa = jnp.concatenate([x, ones_col], axis=1)
    xa_hi, xa_lo = _split(xa)

    def _dot(a, b):
        return jax.lax.dot_general(
            a, b, dimension_numbers=(((1,), (0,)), ((), ())),
            preferred_element_type=jnp.float32)

    x3 = jnp.concatenate([xa_hi, xa_hi, xa_lo, xa_lo], axis=1)
    dist = _dot(x3, c3)

    minval = jnp.min(dist, axis=1, keepdims=True)
    iota_kf = jax.lax.broadcasted_iota(
        jnp.int32, (BN, K), 1).astype(jnp.float32)
    masked = jnp.where(dist == minval, iota_kf, jnp.float32(K))
    idx = jnp.min(masked, axis=1, keepdims=True).astype(jnp.int32)
    assign_ref[...] = idx

    xn = jnp.sum(x * x, axis=1, keepdims=True)
    sacc_scr[...] = sacc_scr[...] + jnp.sum(minval + xn)

    @pl.when(i == NB - 1)
    def _finish():
        sdist_ref[...] = sacc_scr[...]


def _sc_segsum(x_hbm, idx_hbm, z_sums_hbm, z_cnt_hbm, ones_hbm,
               sums_out, cnt_out, x_v, idx_v, ones_v,
               shared_sums, shared_cnt):
    cid = lax.axis_index("c")
    sid = lax.axis_index("s")
    base = (cid * NS + sid) * RPW

    pltpu.sync_copy(ones_hbm, ones_v)

    @pl.when(sid == 0)
    def _zero():
        pltpu.sync_copy(z_sums_hbm, shared_sums)
        pltpu.sync_copy(z_cnt_hbm, shared_cnt)

    plsc.subcore_barrier()

    def _chunk(t, carry):
        b = base + t * CH
        pltpu.sync_copy(x_hbm.at[pl.ds(b, CH)], x_v)
        pltpu.sync_copy(idx_hbm.at[pl.ds(b, CH)], idx_v)
        pltpu.sync_copy(x_v, shared_sums.at[idx_v], add=True)
        pltpu.sync_copy(ones_v, shared_cnt.at[idx_v], add=True)
        return carry

    lax.fori_loop(0, NCH, _chunk, 0)
    plsc.subcore_barrier()

    @pl.when(sid == 0)
    def _flush():
        pltpu.sync_copy(shared_sums, sums_out.at[cid])
        pltpu.sync_copy(shared_cnt, cnt_out.at[cid])


_sc_segsum_call = pl.kernel(
    _sc_segsum,
    out_type=[
        jax.ShapeDtypeStruct((NC, K, D), jnp.float32),
        jax.ShapeDtypeStruct((NC, K, CW), jnp.float32),
    ],
    mesh=plsc.VectorSubcoreMesh(core_axis_name="c", subcore_axis_name="s"),
    scratch_types=[
        pltpu.VMEM((CH, D), jnp.float32),
        pltpu.VMEM((CH,), jnp.int32),
        pltpu.VMEM((CH, CW), jnp.float32),
        pltpu.VMEM_SHARED((K, D), jnp.float32),
        pltpu.VMEM_SHARED((K, CW), jnp.float32),
    ],
)


def _combine_body(sums_ref, cnt_ref, cent_ref):
    s = sums_ref[0, :, :] + sums_ref[1, :, :]
    c = cnt_ref[0, :, :1] + cnt_ref[1, :, :1]
    cent_ref[...] = s / c


@jax.jit
def kernel(input_x, input_centroids):
    assign2, sdist = pl.pallas_call(
        _assign_body,
        grid=(NB,),
        in_specs=[
            pl.BlockSpec((BN, D), lambda i: (i, 0)),
            pl.BlockSpec((D, K), lambda i: (0, 0)),
        ],
        out_specs=[
            pl.BlockSpec((BN, 1), lambda i: (i, 0)),
            pl.BlockSpec((1, 1), lambda i: (0, 0)),
        ],
        out_shape=[
            jax.ShapeDtypeStruct((N, 1), jnp.int32),
            jax.ShapeDtypeStruct((1, 1), jnp.float32),
        ],
        scratch_shapes=[
            pltpu.VMEM((1, 1), jnp.float32),
        ],
    )(input_x, input_centroids.T)
    assignments = assign2.reshape(N)

    sums_p, cnt_p = _sc_segsum_call(
        input_x,
        assignments,
        jnp.zeros((K, D), jnp.float32),
        jnp.zeros((K, CW), jnp.float32),
        jnp.ones((CH, CW), jnp.float32),
    )

    cent = pl.pallas_call(
        _combine_body,
        out_shape=jax.ShapeDtypeStruct((K, D), jnp.float32),
    )(sums_p, cnt_p)

    return assignments, cent, sdist[0, 0]


# speedup vs baseline: 1.2730x; 1.0922x over previous
import jax
import jax.numpy as jnp
from jax import lax
from jax.experimental import pallas as pl
from jax.experimental.pallas import tpu as pltpu
from jax.experimental.pallas import tpu_sc as plsc

N, K, D = 65536, 1024, 32
BN = 8192
NB = N // BN
DA = D + 1

NC, NS, L = 2, 16, 16
NW = NC * NS
RPW = N // NW
CH = 128
NCH = RPW // CH
CW = 8


def _assign_body(x_ref, ct_ref, assign_ref, sdist_ref, sacc_scr):
    i = pl.program_id(0)

    def _split(v):
        hi = v.astype(jnp.bfloat16)
        lo = (v - hi.astype(jnp.float32)).astype(jnp.bfloat16)
        return hi, lo

    @pl.when(i == 0)
    def _init():
        sacc_scr[...] = jnp.zeros_like(sacc_scr)

    ct = ct_ref[...]
    cn = jnp.sum(ct * ct, axis=0, keepdims=True)
    ca = jnp.concatenate([-2.0 * ct, cn], axis=0)
    ca_hi, ca_lo = _split(ca)
    c3 = jnp.concatenate([ca_hi, ca_lo, ca_hi, ca_lo], axis=0)

    x = x_ref[...]
    ones_col = jnp.ones((BN, 1), jnp.float32)
    xa = jnp.concatenate([x, ones_col], axis=1)
    xa_hi, xa_lo = _split(xa)

    def _dot(a, b):
        return jax.lax.dot_general(
            a, b, dimension_numbers=(((1,), (0,)), ((), ())),
            preferred_element_type=jnp.float32)

    x3 = jnp.concatenate([xa_hi, xa_hi, xa_lo, xa_lo], axis=1)
    dist = _dot(x3, c3)

    minval = jnp.min(dist, axis=1, keepdims=True)
    iota_kf = jax.lax.broadcasted_iota(
        jnp.int32, (BN, K), 1).astype(jnp.float32)
    masked = jnp.where(dist == minval, iota_kf, jnp.float32(K))
    idx = jnp.min(masked, axis=1, keepdims=True).astype(jnp.int32)
    assign_ref[...] = idx

    xn = jnp.sum(x * x, axis=1, keepdims=True)
    sacc_scr[...] = sacc_scr[...] + jnp.sum(minval + xn)

    @pl.when(i == NB - 1)
    def _finish():
        sdist_ref[...] = sacc_scr[...]


def _sc_segsum(x_hbm, idx_hbm, z_sums_hbm, z_cnt_hbm, ones_hbm,
               sums_out, cnt_out, x_v0, x_v1, idx_v0, idx_v1, ones_v,
               sem_x0, sem_x1, sem_i0, sem_i1,
               shared_sums, shared_cnt):
    cid = lax.axis_index("c")
    sid = lax.axis_index("s")
    base = (cid * NS + sid) * RPW

    pltpu.sync_copy(ones_hbm, ones_v)

    @pl.when(sid == 0)
    def _zero():
        pltpu.sync_copy(z_sums_hbm, shared_sums)
        pltpu.sync_copy(z_cnt_hbm, shared_cnt)

    plsc.subcore_barrier()

    def _load(t, x_v, idx_v, sem_x, sem_i):
        b = jnp.minimum(base + t * CH, N - CH)
        pltpu.async_copy(x_hbm.at[pl.ds(b, CH)], x_v, sem_x)
        pltpu.async_copy(idx_hbm.at[pl.ds(b, CH)], idx_v, sem_i)

    def _drain(t, x_v, idx_v, sem_x, sem_i):
        b = jnp.minimum(base + t * CH, N - CH)
        pltpu.make_async_copy(x_hbm.at[pl.ds(b, CH)], x_v, sem_x).wait()
        pltpu.make_async_copy(idx_hbm.at[pl.ds(b, CH)], idx_v, sem_i).wait()

    def _scatter(x_v, idx_v):
        pltpu.sync_copy(x_v, shared_sums.at[idx_v], add=True)
        pltpu.sync_copy(ones_v, shared_cnt.at[idx_v], add=True)

    _load(0, x_v0, idx_v0, sem_x0, sem_i0)
    _load(1, x_v1, idx_v1, sem_x1, sem_i1)

    def _pair(j, carry):
        t0 = 2 * j
        _drain(t0, x_v0, idx_v0, sem_x0, sem_i0)
        _scatter(x_v0, idx_v0)
        _load(t0 + 2, x_v0, idx_v0, sem_x0, sem_i0)
        _drain(t0 + 1, x_v1, idx_v1, sem_x1, sem_i1)
        _scatter(x_v1, idx_v1)
        _load(t0 + 3, x_v1, idx_v1, sem_x1, sem_i1)
        return carry

    lax.fori_loop(0, NCH // 2, _pair, 0)
    _drain(NCH, x_v0, idx_v0, sem_x0, sem_i0)
    _drain(NCH + 1, x_v1, idx_v1, sem_x1, sem_i1)
    plsc.subcore_barrier()

    @pl.when(sid == 0)
    def _flush():
        pltpu.sync_copy(shared_sums, sums_out.at[cid])
        pltpu.sync_copy(shared_cnt, cnt_out.at[cid])


_sc_segsum_call = pl.kernel(
    _sc_segsum,
    out_type=[
        jax.ShapeDtypeStruct((NC, K, D), jnp.float32),
        jax.ShapeDtypeStruct((NC, K, CW), jnp.float32),
    ],
    mesh=plsc.VectorSubcoreMesh(core_axis_name="c", subcore_axis_name="s"),
    scratch_types=[
        pltpu.VMEM((CH, D), jnp.float32),
        pltpu.VMEM((CH, D), jnp.float32),
        pltpu.VMEM((CH,), jnp.int32),
        pltpu.VMEM((CH,), jnp.int32),
        pltpu.VMEM((CH, CW), jnp.float32),
        pltpu.SemaphoreType.DMA,
        pltpu.SemaphoreType.DMA,
        pltpu.SemaphoreType.DMA,
        pltpu.SemaphoreType.DMA,
        pltpu.VMEM_SHARED((K, D), jnp.float32),
        pltpu.VMEM_SHARED((K, CW), jnp.float32),
    ],
)


def _combine_body(sums_ref, cnt_ref, cent_ref):
    s = sums_ref[0, :, :] + sums_ref[1, :, :]
    c = cnt_ref[0, :, :1] + cnt_ref[1, :, :1]
    cent_ref[...] = s / c


@jax.jit
def kernel(input_x, input_centroids):
    assign2, sdist = pl.pallas_call(
        _assign_body,
        grid=(NB,),
        in_specs=[
            pl.BlockSpec((BN, D), lambda i: (i, 0)),
            pl.BlockSpec((D, K), lambda i: (0, 0)),
        ],
        out_specs=[
            pl.BlockSpec((BN, 1), lambda i: (i, 0)),
            pl.BlockSpec((1, 1), lambda i: (0, 0)),
        ],
        out_shape=[
            jax.ShapeDtypeStruct((N, 1), jnp.int32),
            jax.ShapeDtypeStruct((1, 1), jnp.float32),
        ],
        scratch_shapes=[
            pltpu.VMEM((1, 1), jnp.float32),
        ],
    )(input_x, input_centroids.T)
    assignments = assign2.reshape(N)

    sums_p, cnt_p = _sc_segsum_call(
        input_x,
        assignments,
        jnp.zeros((K, D), jnp.float32),
        jnp.zeros((K, CW), jnp.float32),
        jnp.ones((CH, CW), jnp.float32),
    )

    cent = pl.pallas_call(
        _combine_body,
        out_shape=jax.ShapeDtypeStruct((K, D), jnp.float32),
    )(sums_p, cnt_p)

    return assignments, cent, sdist[0, 0]
